# Initial kernel scaffold; baseline (speedup 1.0000x reference)
import jax
import jax.numpy as jnp
from jax.experimental import pallas as pl


def kernel(x_repeated, c_samples, edge_index, batch, W1_0, b1_0, W2_0, b2_0, eps_0, W1_1, b1_1, W2_1, b2_1, eps_1, W1_2, b1_2, W2_2, b2_2, eps_2, Wp, bp):
    raise NotImplementedError("write your pallas kernel here")


# speedup vs baseline: 25.2912x; 25.2912x over previous
import functools

import jax
import jax.numpy as jnp
from jax import lax
from jax.experimental import pallas as pl
from jax.experimental.pallas import tpu as pltpu
from jax.experimental.pallas import tpu_sc as plsc

F32 = jnp.float32
HI = lax.Precision.HIGHEST

NCORE = 2
NSUB = 16
EDGE_B = 128


IDX_CHK = 16


def _sc_segsum_body(n_samp, rows_per_sub,
                    u_ref, src_ref, dst_ref, z_ref, out_ref,
                    src_v, dst_v, rows_v, agg_sh, sem):
    c = lax.axis_index("c")
    s = lax.axis_index("s")
    nb = src_ref.shape[1]
    per_core = n_samp // NCORE
    for core in range(NCORE):
        for i in range(per_core):
            ch = core * per_core + i

            @pl.when(c == core)
            def _(ch=ch):
                pltpu.sync_copy(z_ref, agg_sh.at[pl.ds(s * rows_per_sub, rows_per_sub)])
                plsc.subcore_barrier()

                def chunk(k, carry):
                    pltpu.sync_copy(src_ref.at[s, pl.ds(k * IDX_CHK, IDX_CHK)], src_v)
                    pltpu.sync_copy(dst_ref.at[s, pl.ds(k * IDX_CHK, IDX_CHK)], dst_v)

                    def batch(j, carry2):
                        pltpu.async_copy(u_ref.at[ch].at[src_v.at[j]], rows_v, sem).wait()
                        pltpu.sync_copy(rows_v, agg_sh.at[dst_v.at[j]], add=True)
                        return carry2

                    lax.fori_loop(0, IDX_CHK, batch, 0)
                    return carry

                lax.fori_loop(0, nb // IDX_CHK, chunk, 0)
                plsc.subcore_barrier()
                pltpu.sync_copy(
                    agg_sh.at[pl.ds(s * rows_per_sub, rows_per_sub)],
                    out_ref.at[ch].at[pl.ds(s * rows_per_sub, rows_per_sub)])

    return None


def _make_sc_segsum(n_samp, n_nodes, emb, nb, agg_rows, rows_per_sub):
    mesh = plsc.VectorSubcoreMesh(core_axis_name="c", subcore_axis_name="s")
    body = functools.partial(_sc_segsum_body, n_samp, rows_per_sub)
    return pl.kernel(
        body,
        out_type=jax.ShapeDtypeStruct((n_samp, agg_rows, emb), F32),
        mesh=mesh,
        scratch_types=[
            pltpu.VMEM((IDX_CHK, EDGE_B), jnp.int32),
            pltpu.VMEM((IDX_CHK, EDGE_B), jnp.int32),
            pltpu.VMEM((EDGE_B, emb), F32),
            pltpu.VMEM_SHARED((agg_rows, emb), F32),
            pltpu.SemaphoreType.DMA,
        ],
    )


def _mm1_body(in_dim, x_ref, c_ref, w1_ref, out_ref):
    x = x_ref[0]
    cc = c_ref[0]
    u = (jnp.dot(x, w1_ref[:in_dim], precision=HI)
         + jnp.dot(cc, w1_ref[in_dim:], precision=HI))
    out_ref[0] = u


def _combine_body(last, emb, x_ref, s_ref, c_ref, eps_ref, b1_ref, w2_ref,
                  b2_ref, w1n_ref, out_ref):
    u = x_ref[0]
    sg = s_ref[0]
    e = eps_ref[0, 0]
    z = jnp.maximum(e * u + sg + b1_ref[:], 0.0)
    h = jnp.dot(z, w2_ref[:], precision=HI) + b2_ref[:]
    if last:
        out_ref[0] = h
    else:
        h = jnp.maximum(h, 0.0)
        cc = c_ref[0]
        out_ref[0] = (jnp.dot(h, w1n_ref[:emb], precision=HI)
                      + jnp.dot(cc, w1n_ref[emb:], precision=HI))


def _readout_body(num_graphs, h_ref, batch_ref, wp_ref, bp_ref, out_ref):
    n_samp, n, _ = h_ref.shape
    bcol = batch_ref[:]
    gids = lax.broadcasted_iota(jnp.int32, (n, num_graphs), 1)
    onehot = (bcol == gids).astype(F32)
    acc = jnp.zeros(out_ref.shape, F32)
    for sp in range(n_samp):
        hg = lax.dot_general(onehot, h_ref[sp],
                             (((0,), (0,)), ((), ())), precision=HI)
        acc = acc + jnp.dot(hg, wp_ref[:], precision=HI)
    out_ref[:] = acc * (1.0 / n_samp) + bp_ref[:]


def kernel(x_repeated, c_samples, edge_index, batch,
           W1_0, b1_0, W2_0, b2_0, eps_0,
           W1_1, b1_1, W2_1, b2_1, eps_1,
           W1_2, b1_2, W2_2, b2_2, eps_2,
           Wp, bp):
    n, n_samp, in_dim = x_repeated.shape
    col_dim = c_samples.shape[-1]
    emb = W2_0.shape[0]
    num_graphs, num_classes = 64, Wp.shape[1]
    n_edges = edge_index.shape[1]

    per_sub = -(-n_edges // NSUB)
    nb = IDX_CHK * (-(-per_sub // (EDGE_B * IDX_CHK)))
    ep = NSUB * nb * EDGE_B
    rows_per_sub = 8 * (-(-(n + 1) // (8 * NSUB)))
    agg_rows = NSUB * rows_per_sub

    src = edge_index[0]
    dst = edge_index[1]
    pad = ep - n_edges
    src3 = jnp.concatenate([src, jnp.zeros((pad,), jnp.int32)]).reshape(NSUB, nb, EDGE_B)
    dst3 = jnp.concatenate([dst, jnp.full((pad,), n, jnp.int32)]).reshape(NSUB, nb, EDGE_B)
    zrows = jnp.zeros((rows_per_sub, emb), F32)

    sc_segsum = _make_sc_segsum(n_samp, n, emb, nb, agg_rows, rows_per_sub)

    nblk = 5
    blk = n // nblk
    mm1 = pl.pallas_call(
        functools.partial(_mm1_body, in_dim),
        grid=(n_samp, nblk),
        in_specs=[
            pl.BlockSpec((1, blk, in_dim), lambda sp, b: (sp, b, 0)),
            pl.BlockSpec((1, blk, col_dim), lambda sp, b: (sp, b, 0)),
            pl.BlockSpec((in_dim + col_dim, emb), lambda sp, b: (0, 0)),
        ],
        out_specs=pl.BlockSpec((1, blk, emb), lambda sp, b: (sp, b, 0)),
        out_shape=jax.ShapeDtypeStruct((n_samp, n, emb), F32),
    )
    x_t = jnp.transpose(x_repeated, (1, 0, 2))
    c_t = jnp.transpose(c_samples, (1, 0, 2))
    u = mm1(x_t, c_t, W1_0)

    params = [(b1_0, W2_0, b2_0, eps_0, W1_1),
              (b1_1, W2_1, b2_1, eps_1, W1_2),
              (b1_2, W2_2, b2_2, eps_2, W1_2)]
    for l, (b1, W2, b2, eps, W1n) in enumerate(params):
        s_t = sc_segsum(u, src3, dst3, zrows)
        last = l == len(params) - 1
        combine = pl.pallas_call(
            functools.partial(_combine_body, last, emb),
            grid=(n_samp, nblk),
            in_specs=[
                pl.BlockSpec((1, blk, emb), lambda sp, b: (sp, b, 0)),
                pl.BlockSpec((1, blk, emb), lambda sp, b: (sp, b, 0)),
                pl.BlockSpec((1, blk, col_dim), lambda sp, b: (sp, b, 0)),
                pl.BlockSpec((1, 1), lambda sp, b: (0, 0)),
                pl.BlockSpec((1, emb), lambda sp, b: (0, 0)),
                pl.BlockSpec((emb, emb), lambda sp, b: (0, 0)),
                pl.BlockSpec((1, emb), lambda sp, b: (0, 0)),
                pl.BlockSpec((emb + col_dim, emb), lambda sp, b: (0, 0)),
            ],
            out_specs=pl.BlockSpec((1, blk, emb), lambda sp, b: (sp, b, 0)),
            out_shape=jax.ShapeDtypeStruct((n_samp, n, emb), F32),
        )
        u = combine(u, s_t, c_t, jnp.reshape(1.0 + eps, (1, 1)),
                    b1.reshape(1, emb), W2, b2.reshape(1, emb), W1n)

    readout = pl.pallas_call(
        functools.partial(_readout_body, num_graphs),
        out_shape=jax.ShapeDtypeStruct((num_graphs, num_classes), F32),
    )
    return readout(u, batch.reshape(n, 1), Wp, bp.reshape(1, num_classes))


# speedup vs baseline: 30.9065x; 1.2220x over previous
import functools

import jax
import jax.numpy as jnp
from jax import lax
from jax.experimental import pallas as pl
from jax.experimental.pallas import tpu as pltpu
from jax.experimental.pallas import tpu_sc as plsc

F32 = jnp.float32
HI = lax.Precision.HIGHEST

NCORE = 2
NSUB = 16
EDGE_B = 128


IDX_CHK = 32


def _sc_segsum_body(n_samp, rows_per_sub,
                    u_ref, src_ref, dst_ref, z_ref, out_ref,
                    src_v, dst_v, rows_a, rows_b, agg_sh, gsem, ssem):
    c = lax.axis_index("c")
    s = lax.axis_index("s")
    nb = src_ref.shape[1]
    per_core = n_samp // NCORE
    bufs = (rows_a, rows_b)
    for core in range(NCORE):
        for i in range(per_core):
            ch = core * per_core + i

            @pl.when(c == core)
            def _(ch=ch):
                pltpu.sync_copy(z_ref, agg_sh.at[pl.ds(s * rows_per_sub, rows_per_sub)])
                plsc.subcore_barrier()

                def chunk(k, carry):
                    pltpu.sync_copy(src_ref.at[s, pl.ds(k * IDX_CHK, IDX_CHK)], src_v)
                    pltpu.sync_copy(dst_ref.at[s, pl.ds(k * IDX_CHK, IDX_CHK)], dst_v)
                    for b in range(2):
                        pltpu.async_copy(u_ref.at[ch].at[src_v.at[b]], bufs[b], gsem)
                    drain_src = u_ref.at[ch, pl.ds(0, EDGE_B)]

                    def batch(j, carry2):
                        pltpu.make_async_copy(drain_src, rows_a, gsem).wait()
                        for b in range(2):
                            @pl.when(j % 2 == b)
                            def _(b=b):
                                pltpu.async_copy(bufs[b], agg_sh.at[dst_v.at[j]],
                                                 ssem, add=True)
                        pltpu.make_async_copy(drain_src, rows_a, ssem).wait()

                        @pl.when(j + 2 < IDX_CHK)
                        def _():
                            for b in range(2):
                                @pl.when(j % 2 == b)
                                def _(b=b):
                                    pltpu.async_copy(
                                        u_ref.at[ch].at[src_v.at[j + 2]],
                                        bufs[b], gsem)

                        return carry2

                    lax.fori_loop(0, IDX_CHK, batch, 0)
                    return carry

                lax.fori_loop(0, nb // IDX_CHK, chunk, 0)
                plsc.subcore_barrier()
                pltpu.sync_copy(
                    agg_sh.at[pl.ds(s * rows_per_sub, rows_per_sub)],
                    out_ref.at[ch].at[pl.ds(s * rows_per_sub, rows_per_sub)])

    return None


def _make_sc_segsum(n_samp, n_nodes, emb, nb, agg_rows, rows_per_sub):
    mesh = plsc.VectorSubcoreMesh(core_axis_name="c", subcore_axis_name="s")
    body = functools.partial(_sc_segsum_body, n_samp, rows_per_sub)
    return pl.kernel(
        body,
        out_type=jax.ShapeDtypeStruct((n_samp, agg_rows, emb), F32),
        mesh=mesh,
        scratch_types=[
            pltpu.VMEM((IDX_CHK, EDGE_B), jnp.int32),
            pltpu.VMEM((IDX_CHK, EDGE_B), jnp.int32),
            pltpu.VMEM((EDGE_B, emb), F32),
            pltpu.VMEM((EDGE_B, emb), F32),
            pltpu.VMEM_SHARED((agg_rows, emb), F32),
            pltpu.SemaphoreType.DMA,
            pltpu.SemaphoreType.DMA,
        ],
    )


def _mm1_body(in_dim, x_ref, c_ref, w1_ref, out_ref):
    x = x_ref[0]
    cc = c_ref[0]
    u = (jnp.dot(x, w1_ref[:in_dim], precision=HI)
         + jnp.dot(cc, w1_ref[in_dim:], precision=HI))
    out_ref[0] = u


def _combine_body(last, emb, x_ref, s_ref, c_ref, eps_ref, b1_ref, w2_ref,
                  b2_ref, w1n_ref, out_ref):
    u = x_ref[0]
    sg = s_ref[0]
    e = eps_ref[0, 0]
    z = jnp.maximum(e * u + sg + b1_ref[:], 0.0)
    h = jnp.dot(z, w2_ref[:], precision=HI) + b2_ref[:]
    if last:
        out_ref[0] = h
    else:
        h = jnp.maximum(h, 0.0)
        cc = c_ref[0]
        out_ref[0] = (jnp.dot(h, w1n_ref[:emb], precision=HI)
                      + jnp.dot(cc, w1n_ref[emb:], precision=HI))


def _readout_body(num_graphs, h_ref, batch_ref, wp_ref, bp_ref, out_ref):
    n_samp, n, _ = h_ref.shape
    bcol = batch_ref[:]
    gids = lax.broadcasted_iota(jnp.int32, (n, num_graphs), 1)
    onehot = (bcol == gids).astype(F32)
    acc = jnp.zeros(out_ref.shape, F32)
    for sp in range(n_samp):
        hg = lax.dot_general(onehot, h_ref[sp],
                             (((0,), (0,)), ((), ())), precision=HI)
        acc = acc + jnp.dot(hg, wp_ref[:], precision=HI)
    out_ref[:] = acc * (1.0 / n_samp) + bp_ref[:]


def kernel(x_repeated, c_samples, edge_index, batch,
           W1_0, b1_0, W2_0, b2_0, eps_0,
           W1_1, b1_1, W2_1, b2_1, eps_1,
           W1_2, b1_2, W2_2, b2_2, eps_2,
           Wp, bp):
    n, n_samp, in_dim = x_repeated.shape
    col_dim = c_samples.shape[-1]
    emb = W2_0.shape[0]
    num_graphs, num_classes = 64, Wp.shape[1]
    n_edges = edge_index.shape[1]

    per_sub = -(-n_edges // NSUB)
    nb = IDX_CHK * (-(-per_sub // (EDGE_B * IDX_CHK)))
    ep = NSUB * nb * EDGE_B
    rows_per_sub = 8 * (-(-(n + 1) // (8 * NSUB)))
    agg_rows = NSUB * rows_per_sub

    src = edge_index[0]
    dst = edge_index[1]
    pad = ep - n_edges
    src3 = jnp.concatenate([src, jnp.zeros((pad,), jnp.int32)]).reshape(NSUB, nb, EDGE_B)
    dst3 = jnp.concatenate([dst, jnp.full((pad,), n, jnp.int32)]).reshape(NSUB, nb, EDGE_B)
    zrows = jnp.zeros((rows_per_sub, emb), F32)

    sc_segsum = _make_sc_segsum(n_samp, n, emb, nb, agg_rows, rows_per_sub)

    nblk = 5
    blk = n // nblk
    mm1 = pl.pallas_call(
        functools.partial(_mm1_body, in_dim),
        grid=(n_samp, nblk),
        in_specs=[
            pl.BlockSpec((1, blk, in_dim), lambda sp, b: (sp, b, 0)),
            pl.BlockSpec((1, blk, col_dim), lambda sp, b: (sp, b, 0)),
            pl.BlockSpec((in_dim + col_dim, emb), lambda sp, b: (0, 0)),
        ],
        out_specs=pl.BlockSpec((1, blk, emb), lambda sp, b: (sp, b, 0)),
        out_shape=jax.ShapeDtypeStruct((n_samp, n, emb), F32),
    )
    x_t = jnp.transpose(x_repeated, (1, 0, 2))
    c_t = jnp.transpose(c_samples, (1, 0, 2))
    u = mm1(x_t, c_t, W1_0)

    params = [(b1_0, W2_0, b2_0, eps_0, W1_1),
              (b1_1, W2_1, b2_1, eps_1, W1_2),
              (b1_2, W2_2, b2_2, eps_2, W1_2)]
    for l, (b1, W2, b2, eps, W1n) in enumerate(params):
        s_t = sc_segsum(u, src3, dst3, zrows)
        last = l == len(params) - 1
        combine = pl.pallas_call(
            functools.partial(_combine_body, last, emb),
            grid=(n_samp, nblk),
            in_specs=[
                pl.BlockSpec((1, blk, emb), lambda sp, b: (sp, b, 0)),
                pl.BlockSpec((1, blk, emb), lambda sp, b: (sp, b, 0)),
                pl.BlockSpec((1, blk, col_dim), lambda sp, b: (sp, b, 0)),
                pl.BlockSpec((1, 1), lambda sp, b: (0, 0)),
                pl.BlockSpec((1, emb), lambda sp, b: (0, 0)),
                pl.BlockSpec((emb, emb), lambda sp, b: (0, 0)),
                pl.BlockSpec((1, emb), lambda sp, b: (0, 0)),
                pl.BlockSpec((emb + col_dim, emb), lambda sp, b: (0, 0)),
            ],
            out_specs=pl.BlockSpec((1, blk, emb), lambda sp, b: (sp, b, 0)),
            out_shape=jax.ShapeDtypeStruct((n_samp, n, emb), F32),
        )
        u = combine(u, s_t, c_t, jnp.reshape(1.0 + eps, (1, 1)),
                    b1.reshape(1, emb), W2, b2.reshape(1, emb), W1n)

    readout = pl.pallas_call(
        functools.partial(_readout_body, num_graphs),
        out_shape=jax.ShapeDtypeStruct((num_graphs, num_classes), F32),
    )
    return readout(u, batch.reshape(n, 1), Wp, bp.reshape(1, num_classes))


# speedup vs baseline: 49.8968x; 1.6144x over previous
import functools

import jax
import jax.numpy as jnp
from jax import lax
from jax.experimental import pallas as pl
from jax.experimental.pallas import tpu as pltpu
from jax.experimental.pallas import tpu_sc as plsc

F32 = jnp.float32
HI = lax.Precision.HIGHEST

NCORE = 2
NSUB = 16
EDGE_B = 120
NBUF = 3


IDX_CHK = 8


def _sc_segsum_body(n_samp, rows_per_sub,
                    u_ref, src_ref, dst_ref, z_ref, out_ref,
                    src_v, dst_v, rows_a, rows_b, rows_c, agg_sh, gsem, ssem):
    c = lax.axis_index("c")
    s = lax.axis_index("s")
    nb = src_ref.shape[1]
    per_core = n_samp // NCORE
    bufs = (rows_a, rows_b, rows_c)
    for core in range(NCORE):
        for i in range(per_core):
            ch = core * per_core + i

            @pl.when(c == core)
            def _(ch=ch):
                pltpu.sync_copy(z_ref, agg_sh.at[pl.ds(s * rows_per_sub, rows_per_sub)])
                plsc.subcore_barrier()
                drain_src = u_ref.at[ch, pl.ds(0, EDGE_B)]

                def chunk(k, carry):
                    pltpu.sync_copy(src_ref.at[s, pl.ds(k * IDX_CHK, IDX_CHK)], src_v)
                    pltpu.sync_copy(dst_ref.at[s, pl.ds(k * IDX_CHK, IDX_CHK)], dst_v)
                    for b in range(2):
                        pltpu.async_copy(u_ref.at[ch].at[src_v.at[b]], bufs[b], gsem)

                    def batch(j, carry2):
                        pltpu.make_async_copy(drain_src, rows_a, gsem).wait()
                        for b in range(NBUF):
                            @pl.when(j % NBUF == b)
                            def _(b=b):
                                pltpu.async_copy(bufs[b], agg_sh.at[dst_v.at[j]],
                                                 ssem, add=True)

                        @pl.when(j >= 1)
                        def _():
                            pltpu.make_async_copy(drain_src, rows_a, ssem).wait()

                        @pl.when(j + 2 < IDX_CHK)
                        def _():
                            for b in range(NBUF):
                                @pl.when((j + 2) % NBUF == b)
                                def _(b=b):
                                    pltpu.async_copy(
                                        u_ref.at[ch].at[src_v.at[j + 2]],
                                        bufs[b], gsem)

                        return carry2

                    lax.fori_loop(0, IDX_CHK, batch, 0)
                    pltpu.make_async_copy(drain_src, rows_a, ssem).wait()
                    return carry

                lax.fori_loop(0, nb // IDX_CHK, chunk, 0)
                plsc.subcore_barrier()
                pltpu.sync_copy(
                    agg_sh.at[pl.ds(s * rows_per_sub, rows_per_sub)],
                    out_ref.at[ch].at[pl.ds(s * rows_per_sub, rows_per_sub)])

    return None


def _make_sc_segsum(n_samp, n_nodes, emb, nb, agg_rows, rows_per_sub):
    mesh = plsc.VectorSubcoreMesh(core_axis_name="c", subcore_axis_name="s")
    body = functools.partial(_sc_segsum_body, n_samp, rows_per_sub)
    return pl.kernel(
        body,
        out_type=jax.ShapeDtypeStruct((n_samp, agg_rows, emb), F32),
        mesh=mesh,
        scratch_types=[
            pltpu.VMEM((IDX_CHK, EDGE_B), jnp.int32),
            pltpu.VMEM((IDX_CHK, EDGE_B), jnp.int32),
            pltpu.VMEM((EDGE_B, emb), F32),
            pltpu.VMEM((EDGE_B, emb), F32),
            pltpu.VMEM((EDGE_B, emb), F32),
            pltpu.VMEM_SHARED((agg_rows, emb), F32),
            pltpu.SemaphoreType.DMA,
            pltpu.SemaphoreType.DMA,
        ],
    )


def _mm1_body(in_dim, x_ref, c_ref, w1_ref, out_ref):
    x = x_ref[0]
    cc = c_ref[0]
    u = (jnp.dot(x, w1_ref[:in_dim], precision=HI)
         + jnp.dot(cc, w1_ref[in_dim:], precision=HI))
    out_ref[0] = u


def _combine_body(last, emb, x_ref, s_ref, c_ref, eps_ref, b1_ref, w2_ref,
                  b2_ref, w1n_ref, out_ref):
    u = x_ref[0]
    sg = s_ref[0]
    e = eps_ref[0, 0]
    z = jnp.maximum(e * u + sg + b1_ref[:], 0.0)
    h = jnp.dot(z, w2_ref[:], precision=HI) + b2_ref[:]
    if last:
        out_ref[0] = h
    else:
        h = jnp.maximum(h, 0.0)
        cc = c_ref[0]
        out_ref[0] = (jnp.dot(h, w1n_ref[:emb], precision=HI)
                      + jnp.dot(cc, w1n_ref[emb:], precision=HI))


def _readout_body(num_graphs, h_ref, batch_ref, wp_ref, bp_ref, out_ref):
    n_samp, n, _ = h_ref.shape
    bcol = batch_ref[:]
    gids = lax.broadcasted_iota(jnp.int32, (n, num_graphs), 1)
    onehot = (bcol == gids).astype(F32)
    acc = jnp.zeros(out_ref.shape, F32)
    for sp in range(n_samp):
        hg = lax.dot_general(onehot, h_ref[sp],
                             (((0,), (0,)), ((), ())), precision=HI)
        acc = acc + jnp.dot(hg, wp_ref[:], precision=HI)
    out_ref[:] = acc * (1.0 / n_samp) + bp_ref[:]


def kernel(x_repeated, c_samples, edge_index, batch,
           W1_0, b1_0, W2_0, b2_0, eps_0,
           W1_1, b1_1, W2_1, b2_1, eps_1,
           W1_2, b1_2, W2_2, b2_2, eps_2,
           Wp, bp):
    n, n_samp, in_dim = x_repeated.shape
    col_dim = c_samples.shape[-1]
    emb = W2_0.shape[0]
    num_graphs, num_classes = 64, Wp.shape[1]
    n_edges = edge_index.shape[1]

    per_sub = -(-n_edges // NSUB)
    nb = IDX_CHK * (-(-per_sub // (EDGE_B * IDX_CHK)))
    ep = NSUB * nb * EDGE_B
    rows_per_sub = 8 * (-(-(n + 1) // (8 * NSUB)))
    agg_rows = NSUB * rows_per_sub

    src = edge_index[0]
    dst = edge_index[1]
    pad = ep - n_edges
    src3 = jnp.concatenate([src, jnp.zeros((pad,), jnp.int32)]).reshape(NSUB, nb, EDGE_B)
    dst3 = jnp.concatenate([dst, jnp.full((pad,), n, jnp.int32)]).reshape(NSUB, nb, EDGE_B)
    zrows = jnp.zeros((rows_per_sub, emb), F32)

    sc_segsum = _make_sc_segsum(n_samp, n, emb, nb, agg_rows, rows_per_sub)

    nblk = 5
    blk = n // nblk
    mm1 = pl.pallas_call(
        functools.partial(_mm1_body, in_dim),
        grid=(n_samp, nblk),
        in_specs=[
            pl.BlockSpec((1, blk, in_dim), lambda sp, b: (sp, b, 0)),
            pl.BlockSpec((1, blk, col_dim), lambda sp, b: (sp, b, 0)),
            pl.BlockSpec((in_dim + col_dim, emb), lambda sp, b: (0, 0)),
        ],
        out_specs=pl.BlockSpec((1, blk, emb), lambda sp, b: (sp, b, 0)),
        out_shape=jax.ShapeDtypeStruct((n_samp, n, emb), F32),
    )
    x_t = jnp.transpose(x_repeated, (1, 0, 2))
    c_t = jnp.transpose(c_samples, (1, 0, 2))
    u = mm1(x_t, c_t, W1_0)

    params = [(b1_0, W2_0, b2_0, eps_0, W1_1),
              (b1_1, W2_1, b2_1, eps_1, W1_2),
              (b1_2, W2_2, b2_2, eps_2, W1_2)]
    for l, (b1, W2, b2, eps, W1n) in enumerate(params):
        s_t = sc_segsum(u, src3, dst3, zrows)
        last = l == len(params) - 1
        combine = pl.pallas_call(
            functools.partial(_combine_body, last, emb),
            grid=(n_samp, nblk),
            in_specs=[
                pl.BlockSpec((1, blk, emb), lambda sp, b: (sp, b, 0)),
                pl.BlockSpec((1, blk, emb), lambda sp, b: (sp, b, 0)),
                pl.BlockSpec((1, blk, col_dim), lambda sp, b: (sp, b, 0)),
                pl.BlockSpec((1, 1), lambda sp, b: (0, 0)),
                pl.BlockSpec((1, emb), lambda sp, b: (0, 0)),
                pl.BlockSpec((emb, emb), lambda sp, b: (0, 0)),
                pl.BlockSpec((1, emb), lambda sp, b: (0, 0)),
                pl.BlockSpec((emb + col_dim, emb), lambda sp, b: (0, 0)),
            ],
            out_specs=pl.BlockSpec((1, blk, emb), lambda sp, b: (sp, b, 0)),
            out_shape=jax.ShapeDtypeStruct((n_samp, n, emb), F32),
        )
        u = combine(u, s_t, c_t, jnp.reshape(1.0 + eps, (1, 1)),
                    b1.reshape(1, emb), W2, b2.reshape(1, emb), W1n)

    readout = pl.pallas_call(
        functools.partial(_readout_body, num_graphs),
        out_shape=jax.ShapeDtypeStruct((num_graphs, num_classes), F32),
    )
    return readout(u, batch.reshape(n, 1), Wp, bp.reshape(1, num_classes))
